# baseline (speedup 1.0000x reference)
import functools

import jax
import jax.numpy as jnp
from jax import lax
from jax.experimental import pallas as pl
from jax.experimental.pallas import tpu as pltpu
from jax.experimental.pallas import tpu_sc as plsc

N = 10000
E = 320000
H = 128
EPS = 1e-5

NC = 2
NS = 16
NW = NC * NS
CHUNK = 80
NPH = 7
CPP = 18
NCHUNK = NPH * CPP
EPT = NCHUNK * CHUNK
EPAD = NW * EPT
NPAD = 10240
ROWS_PER_TILE = NPAD // NS


def _spmm_sc(y, src2, dst2, zeros):
    mesh = plsc.VectorSubcoreMesh(core_axis_name="c", subcore_axis_name="s")

    @functools.partial(
        pl.kernel,
        out_type=jax.ShapeDtypeStruct((NC * NPAD, H), jnp.float32),
        mesh=mesh,
        scratch_types=[
            [pltpu.VMEM((CPP, CHUNK), jnp.int32) for _ in range(2)],
            [pltpu.VMEM((CPP, CHUNK), jnp.int32) for _ in range(2)],
            [pltpu.VMEM((CHUNK, H), jnp.float32) for _ in range(2)],
            pltpu.VMEM_SHARED((NPAD, H), jnp.float32),
            [pltpu.SemaphoreType.DMA for _ in range(2)],
            [pltpu.SemaphoreType.DMA for _ in range(2)],
            [pltpu.SemaphoreType.DMA for _ in range(2)],
        ],
    )
    def k(y_hbm, src_hbm, dst_hbm, zeros_hbm, out_hbm,
          src_t, dst_t, rows, agg_sh, tsems, gsems, ssems):
        c = lax.axis_index("c")
        s = lax.axis_index("s")
        wid = c * NS + s
        pltpu.sync_copy(zeros_hbm.at[pl.ds(s * ROWS_PER_TILE, ROWS_PER_TILE)],
                        agg_sh.at[pl.ds(s * ROWS_PER_TILE, ROWS_PER_TILE)])
        plsc.subcore_barrier()

        def tload(ph, sl):
            pltpu.async_copy(src_hbm.at[wid * NPH + ph], src_t[sl], tsems[sl])
            pltpu.async_copy(dst_hbm.at[wid * NPH + ph], dst_t[sl], tsems[sl])

        def twait(ph, sl):
            pltpu.make_async_copy(src_hbm.at[wid * NPH + ph], src_t[sl],
                                  tsems[sl]).wait()
            pltpu.make_async_copy(dst_hbm.at[wid * NPH + ph], dst_t[sl],
                                  tsems[sl]).wait()

        def gstart(sl, j, rsl):
            pltpu.async_copy(y_hbm.at[src_t[sl].at[j]], rows[rsl], gsems[rsl])

        def gwait(sl, j, rsl):
            pltpu.make_async_copy(y_hbm.at[src_t[sl].at[j]], rows[rsl],
                                  gsems[rsl]).wait()

        def sstart(sl, j, rsl):
            pltpu.async_copy(rows[rsl], agg_sh.at[dst_t[sl].at[j]], ssems[rsl],
                             add=True)

        def swait(sl, j, rsl):
            pltpu.make_async_copy(rows[rsl], agg_sh.at[dst_t[sl].at[j]],
                                  ssems[rsl]).wait()

        tload(0, 0)
        for ph in range(NPH):
            sl = ph % 2
            twait(ph, sl)
            if ph + 1 < NPH:
                tload(ph + 1, 1 - sl)
            gstart(sl, 0, 0)

            def body(ii, carry, sl=sl):
                j0 = 2 * ii
                gwait(sl, j0, 0)
                sstart(sl, j0, 0)

                @pl.when(ii > 0)
                def _():
                    swait(sl, j0 - 1, 1)

                gstart(sl, j0 + 1, 1)
                gwait(sl, j0 + 1, 1)
                sstart(sl, j0 + 1, 1)
                swait(sl, j0, 0)

                @pl.when(j0 + 2 < CPP)
                def _():
                    gstart(sl, j0 + 2, 0)

                return carry

            lax.fori_loop(0, CPP // 2, body, 0)
            swait(sl, CPP - 1, 1)
        plsc.subcore_barrier()
        pltpu.sync_copy(agg_sh.at[pl.ds(s * ROWS_PER_TILE, ROWS_PER_TILE)],
                        out_hbm.at[pl.ds(c * NPAD + s * ROWS_PER_TILE, ROWS_PER_TILE)])

    return k(y, src2, dst2, zeros)


def _prep_tc(a0, a1, W1, gamma, beta):
    def body(a0_ref, a1_ref, w_ref, g_ref, b_ref, hn_ref):
        agg = a0_ref[...] + a1_ref[...]
        h = jnp.maximum(
            jnp.dot(agg, w_ref[...], preferred_element_type=jnp.float32), 0.0)
        mean = jnp.mean(h, axis=0, keepdims=True)
        var = jnp.mean(jnp.square(h - mean), axis=0, keepdims=True)
        hn_ref[...] = (h - mean) * lax.rsqrt(var + EPS) * g_ref[...] + b_ref[...]

    return pl.pallas_call(
        body,
        out_shape=jax.ShapeDtypeStruct((N, H), jnp.float32),
    )(a0, a1, W1, gamma.reshape(1, H), beta.reshape(1, H))


BM = 256
GRID_M = (N + BM - 1) // BM


def _decode_tc(hn):
    def body(a_ref, b_ref, o_ref):
        o_ref[...] = lax.dot_general(
            a_ref[...], b_ref[...], (((1,), (1,)), ((), ())),
            preferred_element_type=jnp.float32)

    return pl.pallas_call(
        body,
        grid=(GRID_M,),
        in_specs=[
            pl.BlockSpec((BM, H), lambda i: (i, 0)),
            pl.BlockSpec((N, H), lambda i: (0, 0)),
        ],
        out_specs=pl.BlockSpec((BM, N), lambda i: (i, 0)),
        out_shape=jax.ShapeDtypeStruct((N, N), jnp.float32),
    )(hn, hn)


def kernel(y_features, edge_index, W1, gamma, beta):
    pad = EPAD - E
    src2 = jnp.concatenate(
        [edge_index[0], jnp.zeros((pad,), jnp.int32)]).reshape(NW * NPH, CPP, CHUNK)
    dst2 = jnp.concatenate(
        [edge_index[1], jnp.full((pad,), N, jnp.int32)]).reshape(NW * NPH, CPP, CHUNK)
    zeros = jnp.zeros((NPAD, H), jnp.float32)
    agg2 = _spmm_sc(y_features, src2, dst2, zeros)
    hn = _prep_tc(agg2[:N], agg2[NPAD:NPAD + N], W1, gamma, beta)
    return _decode_tc(hn)


# speedup vs baseline: 1.2089x; 1.2089x over previous
import functools

import jax
import jax.numpy as jnp
from jax import lax
from jax.experimental import pallas as pl
from jax.experimental.pallas import tpu as pltpu
from jax.experimental.pallas import tpu_sc as plsc

N = 10000
E = 320000
H = 128
EPS = 1e-5

NC = 2
NS = 16
NW = NC * NS
EPT = E // NW
CHUNK = 80
NCHUNK = EPT // CHUNK
NPAD = 10240
ROWS_PER_TILE = NPAD // NS


def _spmm_sc(y, src2, dst2, zeros):
    mesh = plsc.VectorSubcoreMesh(core_axis_name="c", subcore_axis_name="s")

    @functools.partial(
        pl.kernel,
        out_type=jax.ShapeDtypeStruct((NC * NPAD, H), jnp.float32),
        mesh=mesh,
        scratch_types=[
            pltpu.VMEM((NCHUNK, CHUNK), jnp.int32),
            pltpu.VMEM((NCHUNK, CHUNK), jnp.int32),
            pltpu.VMEM((CHUNK, H), jnp.float32),
            pltpu.VMEM_SHARED((NPAD, H), jnp.float32),
            pltpu.SemaphoreType.DMA,
        ],
    )
    def k(y_hbm, src_hbm, dst_hbm, zeros_hbm, out_hbm, src_t, dst_t, rows,
          agg_sh, sem):
        c = lax.axis_index("c")
        s = lax.axis_index("s")
        wid = c * NS + s
        pltpu.sync_copy(zeros_hbm.at[pl.ds(s * ROWS_PER_TILE, ROWS_PER_TILE)],
                        agg_sh.at[pl.ds(s * ROWS_PER_TILE, ROWS_PER_TILE)])
        pltpu.sync_copy(src_hbm.at[wid], src_t)
        pltpu.sync_copy(dst_hbm.at[wid], dst_t)
        plsc.subcore_barrier()

        def body(i, carry):
            pltpu.async_copy(y_hbm.at[src_t.at[i]], rows, sem).wait()
            pltpu.sync_copy(rows, agg_sh.at[dst_t.at[i]], add=True)
            return carry

        lax.fori_loop(0, NCHUNK, body, 0)
        plsc.subcore_barrier()
        pltpu.sync_copy(agg_sh.at[pl.ds(s * ROWS_PER_TILE, ROWS_PER_TILE)],
                        out_hbm.at[pl.ds(c * NPAD + s * ROWS_PER_TILE, ROWS_PER_TILE)])

    return k(y, src2, dst2, zeros)


BM = 256
GRID_M = (N + BM - 1) // BM
MPAD = GRID_M * BM


def _dense_tc(a0, a1, W1, gamma, beta):
    def body(a0_ref, a1_ref, w_ref, g_ref, b_ref, o_ref, hn_ref):
        i = pl.program_id(0)

        @pl.when(i == 0)
        def _():
            agg = a0_ref[...] + a1_ref[...]
            h = jnp.maximum(
                jnp.dot(agg, w_ref[...], preferred_element_type=jnp.float32),
                0.0)
            mean = jnp.mean(h, axis=0, keepdims=True)
            var = jnp.mean(jnp.square(h - mean), axis=0, keepdims=True)
            hn_ref[pl.ds(0, N), :] = ((h - mean) * lax.rsqrt(var + EPS)
                                      * g_ref[...] + b_ref[...])
            hn_ref[pl.ds(N, MPAD - N), :] = jnp.zeros((MPAD - N, H),
                                                      jnp.float32)

        o_ref[...] = lax.dot_general(
            hn_ref[pl.ds(i * BM, BM), :], hn_ref[pl.ds(0, N), :],
            (((1,), (1,)), ((), ())), preferred_element_type=jnp.float32)

    return pl.pallas_call(
        body,
        grid=(GRID_M,),
        in_specs=[
            pl.BlockSpec((N, H), lambda i: (0, 0)),
            pl.BlockSpec((N, H), lambda i: (0, 0)),
            pl.BlockSpec((H, H), lambda i: (0, 0)),
            pl.BlockSpec((1, H), lambda i: (0, 0)),
            pl.BlockSpec((1, H), lambda i: (0, 0)),
        ],
        out_specs=pl.BlockSpec((BM, N), lambda i: (i, 0)),
        out_shape=jax.ShapeDtypeStruct((N, N), jnp.float32),
        scratch_shapes=[pltpu.VMEM((MPAD, H), jnp.float32)],
    )(a0, a1, W1, gamma.reshape(1, H), beta.reshape(1, H))


def kernel(y_features, edge_index, W1, gamma, beta):
    src2 = edge_index[0].reshape(NW, NCHUNK, CHUNK)
    dst2 = edge_index[1].reshape(NW, NCHUNK, CHUNK)
    zeros = jnp.zeros((NPAD, H), jnp.float32)
    agg2 = _spmm_sc(y_features, src2, dst2, zeros)
    return _dense_tc(agg2[:N], agg2[NPAD:NPAD + N], W1, gamma, beta)
